# baseline (speedup 1.0000x reference)
import functools

import jax
import jax.numpy as jnp
from jax import lax
from jax.experimental import pallas as pl
from jax.experimental.pallas import tpu as pltpu
from jax.experimental.pallas import tpu_sc as plsc

_NC = 2
_NS = 16
_NW = _NC * _NS

_B = 16384
_V = 1000
_D = 64
_DP = 128
_BPW = _B // _NW
_CHUNK = 128
_NCHUNK = _BPW // _CHUNK

_BLK = 2048


def _gather_kernel(idx_hbm, table_hbm, inter_hbm, idx_v, rows_v, sem, sem_out):
    wid = lax.axis_index("s") * _NC + lax.axis_index("c")
    base = wid * _BPW

    pltpu.sync_copy(idx_hbm.at[wid], idx_v)

    gathers = [
        pltpu.async_copy(
            table_hbm.at[idx_v.at[c]],
            rows_v.at[pl.ds(c * _CHUNK, _CHUNK)],
            sem,
        )
        for c in range(_NCHUNK)
    ]
    writes = []
    for c in range(_NCHUNK):
        gathers[c].wait()
        writes.append(
            pltpu.async_copy(
                rows_v.at[pl.ds(c * _CHUNK, _CHUNK)],
                inter_hbm.at[pl.ds(base + c * _CHUNK, _CHUNK)].at[:, pl.ds(0, _D)],
                sem_out,
            )
        )
    for wcp in writes:
        wcp.wait()


@jax.jit
def _gather_run(idx3, table):
    mesh = plsc.VectorSubcoreMesh(core_axis_name="c", subcore_axis_name="s")
    f = functools.partial(
        pl.kernel,
        mesh=mesh,
        out_type=jax.ShapeDtypeStruct((_B, _DP), jnp.float32),
        compiler_params=pltpu.CompilerParams(
            needs_layout_passes=False, use_tc_tiling_on_sc=False
        ),
        scratch_types=[
            pltpu.VMEM((_NCHUNK, _CHUNK), jnp.int32),
            pltpu.VMEM((_BPW, _D), jnp.float32),
            pltpu.SemaphoreType.DMA,
            pltpu.SemaphoreType.DMA,
        ],
    )(_gather_kernel)
    return f(idx3, table)


def _tc_ln_kernel(x_ref, w_ref, b_ref, o_ref):
    x = x_ref[:, : _D]
    mean = jnp.mean(x, axis=-1, keepdims=True)
    var = jnp.mean((x - mean) ** 2, axis=-1, keepdims=True)
    xhat = (x - mean) * lax.rsqrt(var + jnp.float32(1e-5))
    o_ref[...] = xhat * w_ref[...] + b_ref[...]


@jax.jit
def _ln_run(inter, w2d, b2d):
    return pl.pallas_call(
        _tc_ln_kernel,
        grid=(_B // _BLK,),
        in_specs=[
            pl.BlockSpec((_BLK, _DP), lambda i: (i, 0)),
            pl.BlockSpec((1, _D), lambda i: (0, 0)),
            pl.BlockSpec((1, _D), lambda i: (0, 0)),
        ],
        out_specs=pl.BlockSpec((_BLK, _D), lambda i: (i, 0)),
        out_shape=jax.ShapeDtypeStruct((_B, _D), jnp.float32),
    )(inter, w2d, b2d)


def kernel(intent_id, table, ln_weight, ln_bias):
    idx3 = intent_id.astype(jnp.int32).reshape(_NW, _NCHUNK, _CHUNK)
    inter = _gather_run(idx3, table)
    return _ln_run(inter, ln_weight.reshape(1, _D), ln_bias.reshape(1, _D))


# speedup vs baseline: 1.0004x; 1.0004x over previous
import functools

import jax
import jax.numpy as jnp
from jax import lax
from jax.experimental import pallas as pl
from jax.experimental.pallas import tpu as pltpu
from jax.experimental.pallas import tpu_sc as plsc

_NC = 2
_NS = 16
_NW = _NC * _NS

_B = 16384
_V = 1000
_D = 64
_DP = 128
_BPW = _B // _NW
_CHUNK = 128
_NCHUNK = _BPW // _CHUNK

_BLK = 2048


def _gather_kernel(idx_hbm, table_hbm, inter_hbm, idx_v, rows_v, sem, sem_out):
    wid = lax.axis_index("s") * _NC + lax.axis_index("c")
    base = wid * _BPW

    pltpu.sync_copy(idx_hbm.at[wid], idx_v)

    gathers = [
        pltpu.async_copy(
            table_hbm.at[idx_v.at[c]],
            rows_v.at[pl.ds(c * _CHUNK, _CHUNK)],
            sem,
        )
        for c in range(_NCHUNK)
    ]
    writes = []
    for c in range(_NCHUNK):
        gathers[c].wait()
        writes.append(
            pltpu.async_copy(
                rows_v.at[pl.ds(c * _CHUNK, _CHUNK)],
                inter_hbm.at[pl.ds(base + c * _CHUNK, _CHUNK)].at[:, pl.ds(0, _D)],
                sem_out,
            )
        )
    for wcp in writes:
        wcp.wait()


def _gather_run(idx3, table):
    mesh = plsc.VectorSubcoreMesh(core_axis_name="c", subcore_axis_name="s")
    f = functools.partial(
        pl.kernel,
        mesh=mesh,
        out_type=jax.ShapeDtypeStruct((_B, _DP), jnp.float32),
        compiler_params=pltpu.CompilerParams(
            needs_layout_passes=False, use_tc_tiling_on_sc=False
        ),
        scratch_types=[
            pltpu.VMEM((_NCHUNK, _CHUNK), jnp.int32),
            pltpu.VMEM((_BPW, _D), jnp.float32),
            pltpu.SemaphoreType.DMA,
            pltpu.SemaphoreType.DMA,
        ],
    )(_gather_kernel)
    return f(idx3, table)


def _tc_ln_kernel(x_ref, w_ref, b_ref, o_ref):
    x = x_ref[:, : _D]
    mean = jnp.mean(x, axis=-1, keepdims=True)
    var = jnp.mean((x - mean) ** 2, axis=-1, keepdims=True)
    xhat = (x - mean) * lax.rsqrt(var + jnp.float32(1e-5))
    o_ref[...] = xhat * w_ref[...] + b_ref[...]


def _ln_run(inter, w2d, b2d):
    return pl.pallas_call(
        _tc_ln_kernel,
        grid=(_B // _BLK,),
        in_specs=[
            pl.BlockSpec((_BLK, _DP), lambda i: (i, 0)),
            pl.BlockSpec((1, _D), lambda i: (0, 0)),
            pl.BlockSpec((1, _D), lambda i: (0, 0)),
        ],
        out_specs=pl.BlockSpec((_BLK, _D), lambda i: (i, 0)),
        out_shape=jax.ShapeDtypeStruct((_B, _D), jnp.float32),
    )(inter, w2d, b2d)


@jax.jit
def kernel(intent_id, table, ln_weight, ln_bias):
    idx3 = intent_id.astype(jnp.int32).reshape(_NW, _NCHUNK, _CHUNK)
    inter = _gather_run(idx3, table)
    return _ln_run(inter, ln_weight.reshape(1, _D), ln_bias.reshape(1, _D))


# speedup vs baseline: 1.3485x; 1.3480x over previous
import functools

import jax
import jax.numpy as jnp
from jax import lax
from jax.experimental import pallas as pl
from jax.experimental.pallas import tpu as pltpu
from jax.experimental.pallas import tpu_sc as plsc

_NC = 2
_NS = 16
_NW = _NC * _NS

_B = 16384
_V = 1000
_D = 64
_DP = 128
_BPW = _B // _NW
_CHUNK = 128
_NCHUNK = _BPW // _CHUNK

_BLK = 4096


def _gather_kernel(idx_hbm, table_hbm, inter_hbm, idx_v, rows_v, tbl_sh, sem, sem_out):
    sid = lax.axis_index("s")
    wid = sid * _NC + lax.axis_index("c")
    base = wid * _BPW

    @pl.when(sid == 0)
    def _():
        pltpu.sync_copy(table_hbm, tbl_sh)

    pltpu.sync_copy(idx_hbm.at[wid], idx_v)
    plsc.subcore_barrier()

    gathers = [
        pltpu.async_copy(
            tbl_sh.at[idx_v.at[c]],
            rows_v.at[pl.ds(c * _CHUNK, _CHUNK)],
            sem,
        )
        for c in range(_NCHUNK)
    ]
    writes = []
    for c in range(_NCHUNK):
        gathers[c].wait()
        writes.append(
            pltpu.async_copy(
                rows_v.at[pl.ds(c * _CHUNK, _CHUNK)],
                inter_hbm.at[pl.ds(base + c * _CHUNK, _CHUNK)].at[:, pl.ds(0, _D)],
                sem_out,
            )
        )
    for wcp in writes:
        wcp.wait()


def _gather_run(idx3, table):
    mesh = plsc.VectorSubcoreMesh(core_axis_name="c", subcore_axis_name="s")
    f = functools.partial(
        pl.kernel,
        mesh=mesh,
        out_type=jax.ShapeDtypeStruct((_B, _DP), jnp.float32),
        compiler_params=pltpu.CompilerParams(
            needs_layout_passes=False, use_tc_tiling_on_sc=False
        ),
        scratch_types=[
            pltpu.VMEM((_NCHUNK, _CHUNK), jnp.int32),
            pltpu.VMEM((_BPW, _D), jnp.float32),
            pltpu.VMEM_SHARED((_V, _D), jnp.float32),
            pltpu.SemaphoreType.DMA,
            pltpu.SemaphoreType.DMA,
        ],
    )(_gather_kernel)
    return f(idx3, table)


def _tc_ln_kernel(x_ref, w_ref, b_ref, o_ref):
    x = x_ref[:, : _D]
    mean = jnp.mean(x, axis=-1, keepdims=True)
    var = jnp.mean((x - mean) ** 2, axis=-1, keepdims=True)
    xhat = (x - mean) * lax.rsqrt(var + jnp.float32(1e-5))
    o_ref[...] = (xhat * w_ref[...] + b_ref[...]).T


def _ln_run(inter, w2d, b2d):
    out_t = pl.pallas_call(
        _tc_ln_kernel,
        grid=(_B // _BLK,),
        in_specs=[
            pl.BlockSpec((_BLK, _DP), lambda i: (i, 0)),
            pl.BlockSpec((1, _D), lambda i: (0, 0)),
            pl.BlockSpec((1, _D), lambda i: (0, 0)),
        ],
        out_specs=pl.BlockSpec((_D, _BLK), lambda i: (0, i)),
        out_shape=jax.ShapeDtypeStruct((_D, _B), jnp.float32),
    )(inter, w2d, b2d)
    return jnp.transpose(out_t)


@jax.jit
def kernel(intent_id, table, ln_weight, ln_bias):
    idx3 = intent_id.astype(jnp.int32).reshape(_NW, _NCHUNK, _CHUNK)
    inter = _gather_run(idx3, table)
    return _ln_run(inter, ln_weight.reshape(1, _D), ln_bias.reshape(1, _D))
